# Initial kernel scaffold; baseline (speedup 1.0000x reference)
import jax
import jax.numpy as jnp
from jax.experimental import pallas as pl


def kernel(x, edge_index, batch, W1, b1, W2, b2):
    raise NotImplementedError("write your pallas kernel here")


# speedup vs baseline: 3.0168x; 3.0168x over previous
import functools

import jax
import jax.numpy as jnp
from jax import lax
from jax.experimental import pallas as pl
from jax.experimental.pallas import tpu as pltpu
from jax.experimental.pallas import tpu_sc as plsc

N_NODES = 10000
D = 128
G = 128
NC = 2
NS = 16
NW = NC * NS
RPW = 312
TAIL = 16
SUB = 104
NSUB = RPW // SUB

_mesh = plsc.VectorSubcoreMesh(core_axis_name="c", subcore_axis_name="s")


@functools.partial(
    pl.kernel,
    mesh=_mesh,
    out_type=jax.ShapeDtypeStruct((NC, G, D), jnp.float32),
    scratch_types=[
        pltpu.VMEM((RPW, D), jnp.float32),
        pltpu.VMEM((NSUB, SUB), jnp.int32),
        pltpu.VMEM((TAIL, D), jnp.float32),
        pltpu.VMEM((1, TAIL), jnp.int32),
        pltpu.VMEM((G // NS, D), jnp.float32),
        pltpu.VMEM_SHARED((G, D), jnp.float32),
    ],
)
def _segment_sum_sc(x_hbm, batch_hbm, out_hbm, xbuf, idxbuf, xtail, idxtail,
                    rowbuf, acc):
    cid = lax.axis_index("c")
    sid = lax.axis_index("s")
    w = cid * NS + sid
    rows_per_tile = G // NS

    zero = jnp.zeros((16,), jnp.float32)
    for i in range(rows_per_tile):
        for j in range(D // 16):
            rowbuf[i, pl.ds(j * 16, 16)] = zero
    pltpu.sync_copy(rowbuf, acc.at[pl.ds(sid * rows_per_tile, rows_per_tile)])
    plsc.subcore_barrier()

    base = w * RPW
    pltpu.sync_copy(x_hbm.at[pl.ds(base, RPW)], xbuf)
    for j in range(NSUB):
        pltpu.sync_copy(batch_hbm.at[pl.ds(base + j * SUB, SUB)], idxbuf.at[j])
    for j in range(NSUB):
        pltpu.sync_copy(xbuf.at[pl.ds(j * SUB, SUB)], acc.at[idxbuf.at[j]],
                        add=True)

    @pl.when(w == 0)
    def _():
        pltpu.sync_copy(x_hbm.at[pl.ds(NW * RPW, TAIL)], xtail)
        pltpu.sync_copy(batch_hbm.at[pl.ds(NW * RPW, TAIL)], idxtail.at[0])
        pltpu.sync_copy(xtail, acc.at[idxtail.at[0]], add=True)

    plsc.subcore_barrier()

    pltpu.sync_copy(acc.at[pl.ds(sid * rows_per_tile, rows_per_tile)], rowbuf)
    pltpu.sync_copy(rowbuf, out_hbm.at[cid, pl.ds(sid * rows_per_tile,
                                                  rows_per_tile)])


def _mlp_body(parts_ref, w1_ref, b1_ref, w2_ref, b2_ref, out_ref):
    pooled = parts_ref[0] + parts_ref[1]
    h = jnp.dot(pooled, w1_ref[...], preferred_element_type=jnp.float32)
    h = jnp.maximum(h + b1_ref[...], 0.0)
    out_ref[...] = (
        jnp.dot(h, w2_ref[...], preferred_element_type=jnp.float32)
        + b2_ref[...]
    )


def kernel(x, edge_index, batch, W1, b1, W2, b2):
    del edge_index
    parts = _segment_sum_sc(x, batch)
    return pl.pallas_call(
        _mlp_body,
        out_shape=jax.ShapeDtypeStruct((G, W2.shape[1]), jnp.float32),
    )(parts, W1, b1.reshape(1, -1), W2, b2.reshape(1, -1))


# speedup vs baseline: 3.3276x; 1.1030x over previous
import functools

import jax
import jax.numpy as jnp
from jax import lax
from jax.experimental import pallas as pl
from jax.experimental.pallas import tpu as pltpu
from jax.experimental.pallas import tpu_sc as plsc

N_NODES = 10000
D = 128
G = 128
NC = 2
NS = 16
NW = NC * NS
SUB = 104
NSUB = 3
RPW = SUB * NSUB
TAIL = N_NODES - NW * RPW

_mesh = plsc.VectorSubcoreMesh(core_axis_name="c", subcore_axis_name="s")


@functools.partial(
    pl.kernel,
    mesh=_mesh,
    out_type=jax.ShapeDtypeStruct((NC, G, D), jnp.float32),
    scratch_types=[
        pltpu.VMEM((RPW, D), jnp.float32),
        pltpu.VMEM((NSUB, SUB), jnp.int32),
        pltpu.VMEM((TAIL, D), jnp.float32),
        pltpu.VMEM((1, TAIL), jnp.int32),
        pltpu.VMEM((G // NS, D), jnp.float32),
        pltpu.VMEM_SHARED((G, D), jnp.float32),
        pltpu.SemaphoreType.DMA,
        pltpu.SemaphoreType.DMA,
        pltpu.SemaphoreType.DMA,
        pltpu.SemaphoreType.DMA,
        pltpu.SemaphoreType.DMA,
    ],
)
def _segment_sum_sc(x_hbm, batch_hbm, out_hbm, xbuf, idxbuf,
                    xtail, idxtail, rowbuf, acc, sem_i, sem_x0, sem_x1,
                    sem_x2, sem_s):
    cid = lax.axis_index("c")
    sid = lax.axis_index("s")
    w = cid * NS + sid
    rpt = G // NS
    sems_x = (sem_x0, sem_x1, sem_x2)

    base = w * RPW
    c_ids = [
        pltpu.async_copy(batch_hbm.at[pl.ds(base + j * SUB, SUB)],
                         idxbuf.at[j], sem_i)
        for j in range(NSUB)
    ]
    c_x = [
        pltpu.async_copy(x_hbm.at[pl.ds(base + j * SUB, SUB)],
                         xbuf.at[pl.ds(j * SUB, SUB)], sems_x[j])
        for j in range(NSUB)
    ]

    zero = jnp.zeros((16,), jnp.float32)
    for i in range(rpt):
        for j in range(D // 16):
            rowbuf[i, pl.ds(j * 16, 16)] = zero
    pltpu.sync_copy(rowbuf, acc.at[pl.ds(sid * rpt, rpt)])
    plsc.subcore_barrier()

    for c in c_ids:
        c.wait()
    scats = []
    for j in range(NSUB):
        c_x[j].wait()
        scats.append(
            pltpu.async_copy(xbuf.at[pl.ds(j * SUB, SUB)],
                             acc.at[idxbuf.at[j]], sem_s, add=True))

    @pl.when(w == 0)
    def _():
        pltpu.sync_copy(x_hbm.at[pl.ds(NW * RPW, TAIL)], xtail)
        pltpu.sync_copy(batch_hbm.at[pl.ds(NW * RPW, TAIL)], idxtail.at[0])
        pltpu.sync_copy(xtail, acc.at[idxtail.at[0]], add=True)

    for c in scats:
        c.wait()
    plsc.subcore_barrier()

    pltpu.sync_copy(acc.at[pl.ds(sid * rpt, rpt)],
                    out_hbm.at[cid, pl.ds(sid * rpt, rpt)])


def _mlp_body(parts_ref, w1_ref, b1_ref, w2_ref, b2_ref, out_ref):
    pooled = parts_ref[0] + parts_ref[1]
    h = jnp.dot(pooled, w1_ref[...], preferred_element_type=jnp.float32)
    h = jnp.maximum(h + b1_ref[...], 0.0)
    out_ref[...] = (
        jnp.dot(h, w2_ref[...], preferred_element_type=jnp.float32)
        + b2_ref[...]
    )


def kernel(x, edge_index, batch, W1, b1, W2, b2):
    del edge_index
    parts = _segment_sum_sc(x, batch)
    return pl.pallas_call(
        _mlp_body,
        out_shape=jax.ShapeDtypeStruct((G, W2.shape[1]), jnp.float32),
    )(parts, W1, b1.reshape(1, -1), W2, b2.reshape(1, -1))
